# baseline (speedup 1.0000x reference)
import functools

import jax
import jax.numpy as jnp
from jax import lax
from jax.experimental import pallas as pl
from jax.experimental.pallas import tpu as pltpu
from jax.experimental.pallas import tpu_sc as plsc

_B = 4096 * 200
_D = 64
_DP = 128
_NW = 32
_BPW = _B // _NW
_C = 128
_NBUF = 5
_LAG = 4
_NCH = _BPW // _C
_NT = _NCH // _NBUF

_mesh = plsc.VectorSubcoreMesh(core_axis_name="c", subcore_axis_name="s")


@functools.partial(
    pl.kernel,
    out_type=jax.ShapeDtypeStruct((_B, _DP), jnp.float32),
    mesh=_mesh,
    scratch_types=[
        pltpu.VMEM((_NBUF, _C), jnp.int32),
        pltpu.VMEM((_NBUF, _C, _DP), jnp.float32),
        pltpu.SemaphoreType.DMA((_NBUF,)),
        pltpu.SemaphoreType.DMA((_NBUF,)),
        pltpu.SemaphoreType.DMA((_NBUF,)),
    ],
    compiler_params=pltpu.CompilerParams(use_tc_tiling_on_sc=False),
)
def _gather_kernel(ids_hbm, table_hbm, out_hbm, idx_v, rows_v,
                   isem, gsem, osem):
    wid = lax.axis_index("s") * 2 + lax.axis_index("c")
    base = wid * _BPW

    def fire_idx(g, b):
        pltpu.async_copy(ids_hbm.at[pl.ds(base + g * _C, _C)],
                         idx_v.at[b], isem.at[b])

    def wait_idx(b):
        pltpu.make_async_copy(ids_hbm.at[pl.ds(0, _C)], idx_v.at[b],
                              isem.at[b]).wait()

    def fire_gather(b):
        pltpu.async_copy(table_hbm.at[idx_v.at[b]], rows_v.at[b], gsem.at[b])

    def wait_gather(b):
        pltpu.make_async_copy(table_hbm.at[idx_v.at[b]], rows_v.at[b],
                              gsem.at[b]).wait()

    def fire_out(g, b):
        pltpu.async_copy(rows_v.at[b, :, pl.ds(0, _D)],
                         out_hbm.at[pl.ds(base + g * _C, _C), pl.ds(0, _D)],
                         osem.at[b])

    def wait_out(b):
        pltpu.make_async_copy(rows_v.at[b, :, pl.ds(0, _D)],
                              out_hbm.at[pl.ds(0, _C), pl.ds(0, _D)],
                              osem.at[b]).wait()

    def drain(g, b):
        wait_gather(b)
        fire_out(g, b)
        if isinstance(g, int):
            if g + _NBUF < _NCH:
                fire_idx(g + _NBUF, b)
        else:
            @pl.when(g + _NBUF < _NCH)
            def _():
                fire_idx(g + _NBUF, b)

    for b in range(_NBUF):
        fire_idx(b, b)
    for j in range(_NBUF):
        wait_idx(j)
        fire_gather(j)
        if j >= _LAG:
            drain(j - _LAG, j - _LAG)

    def body(t, carry):
        for b in range(_NBUF):
            g = t * _NBUF + b
            wait_out(b)
            wait_idx(b)
            fire_gather(b)
            drain(g - _LAG, (b - _LAG) % _NBUF)
        return carry

    lax.fori_loop(1, _NT, body, 0)

    for j in range(_NCH - _LAG, _NCH):
        drain(j, j % _NBUF)
    for b in range(_NBUF):
        wait_out(b)


_TU = 7808
_TPW = _TU // _NW
_TROWS = 1000064
_TAILR = _TROWS - _TU * 128 // 1


@functools.partial(
    pl.kernel,
    out_type=jax.ShapeDtypeStruct((_TROWS, _DP), jnp.float32),
    mesh=_mesh,
    scratch_types=[
        pltpu.VMEM((2, _D, _DP), jnp.float32),
        pltpu.VMEM((2, _DP, _DP), jnp.float32),
        pltpu.SemaphoreType.DMA((2,)),
        pltpu.SemaphoreType.DMA((2,)),
    ],
    compiler_params=pltpu.CompilerParams(use_tc_tiling_on_sc=True,
                                         needs_layout_passes=False),
)
def _transpose(tt_hbm, tail_hbm, out_hbm, vals, outb, rsem, wsem):
    wid = lax.axis_index("s") * 2 + lax.axis_index("c")
    lanes = lax.iota(jnp.int32, 16)

    def u_of(tp):
        return tp * _NW + wid

    def fire_read(tp, b):
        pltpu.async_copy(tt_hbm.at[:, pl.ds(u_of(tp) * _DP, _DP)],
                         vals.at[b], rsem.at[b])

    def wait_read(b):
        pltpu.make_async_copy(tt_hbm.at[:, pl.ds(0, _DP)], vals.at[b],
                              rsem.at[b]).wait()

    def fire_write(tp, b):
        pltpu.async_copy(outb.at[b], out_hbm.at[pl.ds(u_of(tp) * _DP, _DP)],
                         wsem.at[b])

    def wait_write(b):
        pltpu.make_async_copy(outb.at[b], out_hbm.at[pl.ds(0, _DP)],
                              wsem.at[b]).wait()

    def compute(b):
        def row(i0, carry):
            for r in range(8):
                i = i0 * 8 + r
                col = jnp.full((16,), i, jnp.int32)
                for k in range(4):
                    vec = plsc.load_gather(vals.at[b], [k * 16 + lanes, col])
                    outb[b, i, pl.ds(k * 16, 16)] = vec
            return carry
        lax.fori_loop(0, _DP // 8, row, 0)

    fire_read(0, 0)
    fire_read(1, 1)

    def body(t, carry):
        for b in range(2):
            tp = 2 * t + b
            wait_read(b)

            @pl.when(t >= 1)
            def _():
                wait_write(b)

            compute(b)
            fire_write(tp, b)

            @pl.when(t <= _TPW // 2 - 2)
            def _():
                fire_read(tp + 2, b)
        return carry

    lax.fori_loop(0, _TPW // 2, body, 0)
    wait_write(0)
    wait_write(1)

    @pl.when(wid == 0)
    def _():
        def tchunk(c, carry):
            pltpu.sync_copy(tail_hbm.at[pl.ds(c * _DP, _DP)], outb.at[0])
            pltpu.sync_copy(outb.at[0],
                            out_hbm.at[pl.ds(_TU * _DP + c * _DP, _DP)])
            return carry
        lax.fori_loop(0, _TAILR // _DP, tchunk, 0)


def kernel(item_ids, table):
    num_embeddings, d = table.shape
    ids = jnp.clip(item_ids.reshape(-1), 0, num_embeddings - 1)
    tail = jnp.pad(table[_TU * 128:],
                   ((0, _TROWS - num_embeddings), (0, _DP - d)))
    table_p = _transpose(table.T, tail)
    out_p = _gather_kernel(ids, table_p)
    return out_p[:, :d].reshape(item_ids.shape + (d,))


# speedup vs baseline: 1.9722x; 1.9722x over previous
import functools

import jax
import jax.numpy as jnp
from jax import lax
from jax.experimental import pallas as pl
from jax.experimental.pallas import tpu as pltpu
from jax.experimental.pallas import tpu_sc as plsc

_B = 4096 * 200
_D = 64
_DP = 128
_NW = 32
_BPW = _B // _NW
_C = 128
_NBUF = 5
_LAG = 4
_NCH = _BPW // _C
_NT = _NCH // _NBUF

_mesh = plsc.VectorSubcoreMesh(core_axis_name="c", subcore_axis_name="s")


@functools.partial(
    pl.kernel,
    out_type=jax.ShapeDtypeStruct((_B, _DP), jnp.float32),
    mesh=_mesh,
    scratch_types=[
        pltpu.VMEM((_NBUF, _C), jnp.int32),
        pltpu.VMEM((_NBUF, _C, _DP), jnp.float32),
        pltpu.SemaphoreType.DMA((_NBUF,)),
        pltpu.SemaphoreType.DMA((_NBUF,)),
        pltpu.SemaphoreType.DMA((_NBUF,)),
    ],
    compiler_params=pltpu.CompilerParams(use_tc_tiling_on_sc=False),
)
def _gather_kernel(ids_hbm, table_hbm, out_hbm, idx_v, rows_v,
                   isem, gsem, osem):
    wid = lax.axis_index("s") * 2 + lax.axis_index("c")
    base = wid * _BPW

    def fire_idx(g, b):
        pltpu.async_copy(ids_hbm.at[pl.ds(base + g * _C, _C)],
                         idx_v.at[b], isem.at[b])

    def wait_idx(b):
        pltpu.make_async_copy(ids_hbm.at[pl.ds(0, _C)], idx_v.at[b],
                              isem.at[b]).wait()

    def fire_gather(b):
        pltpu.async_copy(table_hbm.at[idx_v.at[b]], rows_v.at[b], gsem.at[b])

    def wait_gather(b):
        pltpu.make_async_copy(table_hbm.at[idx_v.at[b]], rows_v.at[b],
                              gsem.at[b]).wait()

    def fire_out(g, b):
        pltpu.async_copy(rows_v.at[b, :, pl.ds(0, _D)],
                         out_hbm.at[pl.ds(base + g * _C, _C), pl.ds(0, _D)],
                         osem.at[b])

    def wait_out(b):
        pltpu.make_async_copy(rows_v.at[b, :, pl.ds(0, _D)],
                              out_hbm.at[pl.ds(0, _C), pl.ds(0, _D)],
                              osem.at[b]).wait()

    def drain(g, b):
        wait_gather(b)
        fire_out(g, b)
        if isinstance(g, int):
            if g + _NBUF < _NCH:
                fire_idx(g + _NBUF, b)
        else:
            @pl.when(g + _NBUF < _NCH)
            def _():
                fire_idx(g + _NBUF, b)

    for b in range(_NBUF):
        fire_idx(b, b)
    for j in range(_NBUF):
        wait_idx(j)
        fire_gather(j)
        if j >= _LAG:
            drain(j - _LAG, j - _LAG)

    def body(t, carry):
        for b in range(_NBUF):
            g = t * _NBUF + b
            wait_out(b)
            wait_idx(b)
            fire_gather(b)
            drain(g - _LAG, (b - _LAG) % _NBUF)
        return carry

    lax.fori_loop(1, _NT, body, 0)

    for j in range(_NCH - _LAG, _NCH):
        drain(j, j % _NBUF)
    for b in range(_NBUF):
        wait_out(b)


def kernel(item_ids, table):
    num_embeddings, d = table.shape
    ids = jnp.clip(item_ids.reshape(-1), 0, num_embeddings - 1)
    table_p = jnp.pad(table, ((0, 0), (0, _DP - d)))
    out_p = _gather_kernel(ids, table_p)
    return out_p[:, :d].reshape(item_ids.shape + (d,))
